# baseline (speedup 1.0000x reference)
import functools

import jax
import jax.numpy as jnp
import numpy as np
from jax import lax
from jax.experimental import pallas as pl
from jax.experimental.pallas import tpu as pltpu
from jax.experimental.pallas import tpu_sc as plsc

N = 10000
E = 320000
DIN = 128
HID = 256
HEADS = 8
FP = 32
L = 6

NC, NS, LANES = 2, 16, 16
NW = NC * NS
NPW = 320
NP = NW * NPW
K = 64
E2 = E + N
E2P = E2 + 240
NBLK = HID // LANES

_SIDX = np.zeros((HID,), np.int32)
for _p in range(HID):
    _i, _l = _p // LANES, _p % LANES
    _f = 2 * _i if _l < 8 else 2 * _i + 1
    _h = _l if _l < 8 else 15 - _l
    _SIDX[_p] = _h * FP + _f
_INV_SIDX = np.argsort(_SIDX).astype(np.int32)


def _leaky(x, s=0.1):
    return jnp.where(x > 0, x, s * x)


def _batchnorm(x, g, b):
    m = jnp.mean(x, axis=0)
    v = jnp.var(x, axis=0)
    return (x - m) / jnp.sqrt(v + 1e-5) * g + b


def _layernorm(x, g, b):
    m = jnp.mean(x, axis=-1, keepdims=True)
    v = jnp.var(x, axis=-1, keepdims=True)
    return (x - m) / jnp.sqrt(v + 1e-5) * g + b


def _matmul_kernel(x_ref, w_ref, b_ref, o_ref):
    o_ref[...] = (
        jnp.dot(x_ref[...], w_ref[...], preferred_element_type=jnp.float32)
        + b_ref[...]
    )


def _matmul(x, w, b):
    m, k = x.shape
    n = w.shape[1]
    bm = 1000
    return pl.pallas_call(
        _matmul_kernel,
        grid=(m // bm,),
        in_specs=[
            pl.BlockSpec((bm, k), lambda i: (i, 0)),
            pl.BlockSpec((k, n), lambda i: (0, 0)),
            pl.BlockSpec((n,), lambda i: (0,)),
        ],
        out_specs=pl.BlockSpec((bm, n), lambda i: (i, 0)),
        out_shape=jax.ShapeDtypeStruct((m, n), jnp.float32),
    )(x, w, b)


def _sc_edge_body(xl, xr, srcs, dsts, wb, attv, out,
                  wb_v, att_v, idx_src, idx_dst, xl_buf, xr_buf, stage,
                  sem_g, sem_o):
    w = lax.axis_index("s") * NC + lax.axis_index("c")
    nd0 = w * NPW

    pltpu.sync_copy(wb.at[w], wb_v)
    pltpu.sync_copy(attv, att_v)

    wrow = wb_v[...]
    e0 = wrow[0]
    e1 = wrow[1]
    a0 = pl.multiple_of((e0 // 8) * 8, 8)
    nch = (e1 - a0 + (K - 1)) // K

    att_regs = [att_v[pl.ds(LANES * i, LANES)] for i in range(NBLK)]

    def issue(ci, buf):
        base = pl.multiple_of(a0 + ci * K, 8)
        pltpu.sync_copy(srcs.at[pl.ds(base, K)], idx_src.at[buf])
        pltpu.sync_copy(dsts.at[pl.ds(base, K)], idx_dst.at[buf])
        pltpu.make_async_copy(
            xl.at[idx_src.at[buf]], xl_buf.at[buf], sem_g.at[buf, 0]).start()
        pltpu.make_async_copy(
            xr.at[idx_dst.at[buf]], xr_buf.at[buf], sem_g.at[buf, 1]).start()

    def finalize(d, acc, asum):
        ln = d - nd0
        r = lax.rem(ln, 4)
        roff = pl.multiple_of(r * HID, 8)

        @pl.when(ln >= 4)
        def _wait_slot():
            pltpu.make_async_copy(
                stage.at[pl.ds(roff, HID)], out.at[d], sem_o.at[r]).wait()

        inv = 1.0 / (asum + 1e-16)
        for i in range(NBLK):
            stage[pl.ds(roff + LANES * i, LANES)] = acc[i] * inv
        pltpu.make_async_copy(
            stage.at[pl.ds(roff, HID)], out.at[d], sem_o.at[r]).start()

    zero = jnp.zeros((LANES,), jnp.float32)

    issue(0, 0)
    issue(1, 1)

    def chunk_body(ci, carry):
        buf = lax.rem(ci, 2)
        base = a0 + ci * K
        pltpu.make_async_copy(
            xl.at[idx_src.at[buf]], xl_buf.at[buf], sem_g.at[buf, 0]).wait()
        pltpu.make_async_copy(
            xr.at[idx_dst.at[buf]], xr_buf.at[buf], sem_g.at[buf, 1]).wait()

        def group_body(g, carry2):
            dvec = idx_dst[buf, pl.ds(LANES * g, LANES)]
            for jj in range(LANES):
                cur_d, acc, asum = carry2
                j = LANES * g + jj
                e = base + j
                valid = jnp.logical_and(e >= e0, e < e1)
                d_j = dvec[jj]
                is_new = jnp.logical_and(valid, d_j != cur_d)

                @pl.when(is_new)
                def _fin(cur_d=cur_d, acc=acc, asum=asum):
                    finalize(cur_d, acc, asum)

                acc = [jnp.where(is_new, zero, a) for a in acc]
                asum = jnp.where(is_new, zero, asum)
                cur_d = jnp.where(is_new, d_j, cur_d)

                xlr = [xl_buf[buf, j, pl.ds(LANES * i, LANES)]
                       for i in range(NBLK)]
                xrr = [xr_buf[buf, j, pl.ds(LANES * i, LANES)]
                       for i in range(NBLK)]
                t = []
                for i in range(NBLK):
                    m = xlr[i] + xrr[i]
                    ee = jnp.maximum(m, 0.2 * m)
                    t.append(ee * att_regs[i])
                while len(t) > 1:
                    t = [t[2 * i2] + t[2 * i2 + 1] for i2 in range(len(t) // 2)]
                alpha = t[0] + jnp.flip(t[0])
                wv = jnp.exp(alpha) * jnp.where(valid, 1.0, 0.0)
                acc = [acc[i] + wv * xlr[i] for i in range(NBLK)]
                asum = asum + wv
                carry2 = (cur_d, acc, asum)
            return carry2

        carry = lax.fori_loop(0, K // LANES, group_body, carry)

        @pl.when(ci + 2 < nch)
        def _issue_next():
            issue(ci + 2, buf)

        return carry

    acc0 = [zero for _ in range(NBLK)]
    asum0 = zero
    cur_d, acc, asum = lax.fori_loop(0, nch, chunk_body, (nd0, acc0, asum0))
    finalize(cur_d, acc, asum)
    for r in range(4):
        pltpu.make_async_copy(
            stage.at[pl.ds(r * HID, HID)], out.at[nd0], sem_o.at[r]).wait()


@functools.partial(
    pl.kernel,
    mesh=plsc.VectorSubcoreMesh(core_axis_name="c", subcore_axis_name="s"),
    out_type=jax.ShapeDtypeStruct((NP, HID), jnp.float32),
    compiler_params=pltpu.CompilerParams(needs_layout_passes=False),
    scratch_types=[
        pltpu.VMEM((LANES,), jnp.int32),
        pltpu.VMEM((HID,), jnp.float32),
        pltpu.VMEM((2, K), jnp.int32),
        pltpu.VMEM((2, K), jnp.int32),
        pltpu.VMEM((2, K, HID), jnp.float32),
        pltpu.VMEM((2, K, HID), jnp.float32),
        pltpu.VMEM((4 * HID,), jnp.float32),
        pltpu.SemaphoreType.DMA((2, 2)),
        pltpu.SemaphoreType.DMA((4,)),
    ],
)
def _sc_edge(xl, xr, srcs, dsts, wb, attv, out,
             wb_v, att_v, idx_src, idx_dst, xl_buf, xr_buf, stage,
             sem_g, sem_o):
    _sc_edge_body(xl, xr, srcs, dsts, wb, attv, out,
                  wb_v, att_v, idx_src, idx_dst, xl_buf, xr_buf, stage,
                  sem_g, sem_o)


def kernel(x, edge_index, W_in, b_in, bn_in_g, bn_in_b, Wl, bl, Wr, br, att,
           gat_bias, ln_g, ln_b, scales, scale_weights, W1, b1, bn1_g, bn1_b,
           W2, b2, W3, b3):
    loop = jnp.arange(N, dtype=edge_index.dtype)
    src = jnp.concatenate([edge_index[0], loop]).astype(jnp.int32)
    dst = jnp.concatenate([edge_index[1], loop]).astype(jnp.int32)
    perm = jnp.argsort(dst)
    srcs = src[perm]
    dsts = dst[perm]
    srcs_p = jnp.pad(srcs, (0, E2P - E2))
    dsts_p = jnp.pad(dsts, (0, E2P - E2))
    wb = jnp.searchsorted(dsts, jnp.arange(33, dtype=jnp.int32) * NPW
                          ).astype(jnp.int32)
    wtab = jnp.zeros((NW, LANES), jnp.int32)
    wtab = wtab.at[:, 0].set(wb[:NW]).at[:, 1].set(wb[1:NW + 1])

    sidx = jnp.asarray(_SIDX)
    inv_sidx = jnp.asarray(_INV_SIDX)
    Wl_p = Wl[:, :, sidx]
    bl_p = bl[:, sidx]
    Wr_p = Wr[:, :, sidx]
    br_p = br[:, sidx]
    att_p = att.reshape(L, HID)[:, sidx]

    h = _leaky(_batchnorm(_matmul(x, W_in, b_in), bn_in_g, bn_in_b))
    w = jax.nn.softmax(scale_weights)
    hacc = jnp.zeros_like(h)
    for i in range(L):
        hn = _layernorm(h, ln_g[i], ln_b[i])
        xl = _matmul(hn, Wl_p[i], bl_p[i])
        xr = _matmul(hn, Wr_p[i], br_p[i])
        gat = _sc_edge(xl, xr, srcs_p, dsts_p, wtab, att_p[i])
        hg = _leaky(gat[:N, inv_sidx] + gat_bias[i])
        h = h + scales[i] * hg
        hacc = hacc + w[i] * h
    h = _leaky(_batchnorm(_matmul(hacc, W1, b1), bn1_g, bn1_b))
    h = _leaky(h @ W2 + b2)
    return h @ W3 + b3


# speedup vs baseline: 1.6548x; 1.6548x over previous
import functools

import jax
import jax.numpy as jnp
import numpy as np
from jax import lax
from jax.experimental import pallas as pl
from jax.experimental.pallas import tpu as pltpu
from jax.experimental.pallas import tpu_sc as plsc

N = 10000
E = 320000
DIN = 128
HID = 256
HEADS = 8
FP = 32
L = 6

NC, NS, LANES = 2, 16, 16
NW = NC * NS
NPW = 320
NP = NW * NPW
K = 64
E2 = E + N
E2P = E2 + 240
NBLK = HID // LANES

_SIDX = np.zeros((HID,), np.int32)
for _p in range(HID):
    _i, _l = _p // LANES, _p % LANES
    _f = 2 * _i if _l < 8 else 2 * _i + 1
    _h = _l if _l < 8 else 15 - _l
    _SIDX[_p] = _h * FP + _f
_INV_SIDX = np.argsort(_SIDX).astype(np.int32)


def _leaky(x, s=0.1):
    return jnp.where(x > 0, x, s * x)


def _batchnorm(x, g, b):
    m = jnp.mean(x, axis=0)
    v = jnp.var(x, axis=0)
    return (x - m) / jnp.sqrt(v + 1e-5) * g + b


def _layernorm(x, g, b):
    m = jnp.mean(x, axis=-1, keepdims=True)
    v = jnp.var(x, axis=-1, keepdims=True)
    return (x - m) / jnp.sqrt(v + 1e-5) * g + b


def _matmul_kernel(x_ref, w_ref, b_ref, o_ref):
    o_ref[...] = (
        jnp.dot(x_ref[...], w_ref[...], preferred_element_type=jnp.float32)
        + b_ref[...]
    )


def _matmul(x, w, b):
    m, k = x.shape
    n = w.shape[1]
    bm = 1000
    return pl.pallas_call(
        _matmul_kernel,
        grid=(m // bm,),
        in_specs=[
            pl.BlockSpec((bm, k), lambda i: (i, 0)),
            pl.BlockSpec((k, n), lambda i: (0, 0)),
            pl.BlockSpec((n,), lambda i: (0,)),
        ],
        out_specs=pl.BlockSpec((bm, n), lambda i: (i, 0)),
        out_shape=jax.ShapeDtypeStruct((m, n), jnp.float32),
    )(x, w, b)


def _sc_edge_body(xl, xr, srcs, dsts, wb, attv, out,
                  wb_v, att_v, idx_src, idx_dst, xl_buf, xr_buf, stage,
                  sem_g, sem_o):
    w = lax.axis_index("s") * NC + lax.axis_index("c")
    nd0 = w * NPW

    pltpu.sync_copy(wb.at[w], wb_v)
    pltpu.sync_copy(attv, att_v)

    wrow = wb_v[...]
    e0 = wrow[0]
    e1 = wrow[1]
    a0 = pl.multiple_of((e0 // 8) * 8, 8)
    nch = (e1 - a0 + (K - 1)) // K

    att_regs = [att_v[pl.ds(LANES * i, LANES)] for i in range(NBLK)]

    def issue(ci, buf):
        base = pl.multiple_of(a0 + ci * K, 8)
        pltpu.sync_copy(srcs.at[pl.ds(base, K)], idx_src.at[buf])
        pltpu.sync_copy(dsts.at[pl.ds(base, K)], idx_dst.at[buf])
        pltpu.make_async_copy(
            xl.at[idx_src.at[buf]], xl_buf.at[buf], sem_g.at[buf, 0]).start()
        pltpu.make_async_copy(
            xr.at[idx_dst.at[buf]], xr_buf.at[buf], sem_g.at[buf, 1]).start()

    def finalize(d, acc, asum):
        ln = d - nd0
        r = lax.rem(ln, 4)
        roff = pl.multiple_of(r * HID, 8)

        @pl.when(ln >= 4)
        def _wait_slot():
            pltpu.make_async_copy(
                stage.at[pl.ds(roff, HID)], out.at[d], sem_o.at[r]).wait()

        inv = 1.0 / (asum + 1e-16)
        for i in range(NBLK):
            stage[pl.ds(roff + LANES * i, LANES)] = acc[i] * inv
        pltpu.make_async_copy(
            stage.at[pl.ds(roff, HID)], out.at[d], sem_o.at[r]).start()

    zero = jnp.zeros((LANES,), jnp.float32)

    issue(0, 0)
    issue(1, 1)

    def chunk_body(ci, carry):
        buf = lax.rem(ci, 2)
        base = a0 + ci * K
        pltpu.make_async_copy(
            xl.at[idx_src.at[buf]], xl_buf.at[buf], sem_g.at[buf, 0]).wait()
        pltpu.make_async_copy(
            xr.at[idx_dst.at[buf]], xr_buf.at[buf], sem_g.at[buf, 1]).wait()

        def group_body(g, carry2):
            dvec = idx_dst[buf, pl.ds(LANES * g, LANES)]
            for jj in range(LANES):
                cur_d, acc, asum = carry2
                j = LANES * g + jj
                e = base + j
                valid = jnp.logical_and(e >= e0, e < e1)
                d_j = dvec[jj]
                is_new = jnp.logical_and(valid, d_j != cur_d)

                @pl.when(is_new)
                def _fin(cur_d=cur_d, acc=acc, asum=asum):
                    finalize(cur_d, acc, asum)

                acc = [jnp.where(is_new, zero, a) for a in acc]
                asum = jnp.where(is_new, zero, asum)
                cur_d = jnp.where(is_new, d_j, cur_d)

                xlr = [xl_buf[buf, j, pl.ds(LANES * i, LANES)]
                       for i in range(NBLK)]
                xrr = [xr_buf[buf, j, pl.ds(LANES * i, LANES)]
                       for i in range(NBLK)]
                t = []
                for i in range(NBLK):
                    m = xlr[i] + xrr[i]
                    ee = jnp.maximum(m, 0.2 * m)
                    t.append(ee * att_regs[i])
                while len(t) > 1:
                    t = [t[2 * i2] + t[2 * i2 + 1] for i2 in range(len(t) // 2)]
                alpha = t[0] + jnp.flip(t[0])
                wv = jnp.exp(alpha) * jnp.where(valid, 1.0, 0.0)
                acc = [acc[i] + wv * xlr[i] for i in range(NBLK)]
                asum = asum + wv
                carry2 = (cur_d, acc, asum)
            return carry2

        carry = lax.fori_loop(0, K // LANES, group_body, carry)

        @pl.when(ci + 2 < nch)
        def _issue_next():
            issue(ci + 2, buf)

        return carry

    acc0 = [zero for _ in range(NBLK)]
    asum0 = zero
    cur_d, acc, asum = lax.fori_loop(0, nch, chunk_body, (nd0, acc0, asum0))
    finalize(cur_d, acc, asum)
    for r in range(4):
        pltpu.make_async_copy(
            stage.at[pl.ds(r * HID, HID)], out.at[nd0], sem_o.at[r]).wait()


@functools.partial(
    pl.kernel,
    mesh=plsc.VectorSubcoreMesh(core_axis_name="c", subcore_axis_name="s"),
    out_type=jax.ShapeDtypeStruct((NP, HID), jnp.float32),
    compiler_params=pltpu.CompilerParams(needs_layout_passes=False),
    scratch_types=[
        pltpu.VMEM((LANES,), jnp.int32),
        pltpu.VMEM((HID,), jnp.float32),
        pltpu.VMEM((2, K), jnp.int32),
        pltpu.VMEM((2, K), jnp.int32),
        pltpu.VMEM((2, K, HID), jnp.float32),
        pltpu.VMEM((2, K, HID), jnp.float32),
        pltpu.VMEM((4 * HID,), jnp.float32),
        pltpu.SemaphoreType.DMA((2, 2)),
        pltpu.SemaphoreType.DMA((4,)),
    ],
)
def _sc_edge(xl, xr, srcs, dsts, wb, attv, out,
             wb_v, att_v, idx_src, idx_dst, xl_buf, xr_buf, stage,
             sem_g, sem_o):
    _sc_edge_body(xl, xr, srcs, dsts, wb, attv, out,
                  wb_v, att_v, idx_src, idx_dst, xl_buf, xr_buf, stage,
                  sem_g, sem_o)


def kernel(x, edge_index, W_in, b_in, bn_in_g, bn_in_b, Wl, bl, Wr, br, att,
           gat_bias, ln_g, ln_b, scales, scale_weights, W1, b1, bn1_g, bn1_b,
           W2, b2, W3, b3):
    loop = jnp.arange(N, dtype=edge_index.dtype)
    src = jnp.concatenate([edge_index[0], loop]).astype(jnp.int32)
    dst = jnp.concatenate([edge_index[1], loop]).astype(jnp.int32)
    perm = jnp.argsort(dst)
    srcs = src[perm]
    dsts = dst[perm]
    srcs_p = jnp.pad(srcs, (0, E2P - E2))
    dsts_p = jnp.pad(dsts, (0, E2P - E2))
    wb = jnp.searchsorted(dsts, jnp.arange(33, dtype=jnp.int32) * NPW
                          ).astype(jnp.int32)
    wtab = jnp.zeros((NW, LANES), jnp.int32)
    wtab = wtab.at[:, 0].set(wb[:NW]).at[:, 1].set(wb[1:NW + 1])

    sidx = jnp.asarray(_SIDX)
    S = jax.nn.one_hot(sidx, HID, axis=0, dtype=jnp.float32)
    W_in_p = W_in @ S
    Wl_pp = jnp.einsum('sp,lst,tq->lpq', S, Wl, S)
    Wr_pp = jnp.einsum('sp,lst,tq->lpq', S, Wr, S)
    W1_pp = S.T @ W1

    h = _leaky(_batchnorm(_matmul(x, W_in_p, b_in[sidx]),
                          bn_in_g[sidx], bn_in_b[sidx]))
    w = jax.nn.softmax(scale_weights)
    hacc = jnp.zeros_like(h)
    for i in range(L):
        hn = _layernorm(h, ln_g[i][sidx], ln_b[i][sidx])
        xl = _matmul(hn, Wl_pp[i], bl[i][sidx])
        xr = _matmul(hn, Wr_pp[i], br[i][sidx])
        gat = _sc_edge(xl, xr, srcs_p, dsts_p, wtab, att[i].reshape(-1)[sidx])
        hg = _leaky(gat[:N] + gat_bias[i][sidx])
        h = h + scales[i] * hg
        hacc = hacc + w[i] * h
    h = _leaky(_batchnorm(_matmul(hacc, W1_pp, b1), bn1_g, bn1_b))
    h = _leaky(h @ W2 + b2)
    return h @ W3 + b3


# speedup vs baseline: 3.2749x; 1.9791x over previous
import functools

import jax
import jax.numpy as jnp
import numpy as np
from jax import lax
from jax.experimental import pallas as pl
from jax.experimental.pallas import tpu as pltpu
from jax.experimental.pallas import tpu_sc as plsc

N = 10000
E = 320000
DIN = 128
HID = 256
HEADS = 8
FP = 32
L = 6

NC, NS, LANES = 2, 16, 16
NW = NC * NS
NPW = 320
NP = NW * NPW
K = 32
E2 = E + N
E2P = E2 + 240
NBLK = HID // LANES

_SIDX = np.zeros((HID,), np.int32)
for _p in range(HID):
    _i, _l = _p // LANES, _p % LANES
    _f = 2 * _i if _l < 8 else 2 * _i + 1
    _h = _l if _l < 8 else 15 - _l
    _SIDX[_p] = _h * FP + _f
_INV_SIDX = np.argsort(_SIDX).astype(np.int32)


def _leaky(x, s=0.1):
    return jnp.where(x > 0, x, s * x)


def _batchnorm(x, g, b):
    m = jnp.mean(x, axis=0)
    v = jnp.var(x, axis=0)
    return (x - m) / jnp.sqrt(v + 1e-5) * g + b


def _layernorm(x, g, b):
    m = jnp.mean(x, axis=-1, keepdims=True)
    v = jnp.var(x, axis=-1, keepdims=True)
    return (x - m) / jnp.sqrt(v + 1e-5) * g + b


def _matmul_kernel(x_ref, w_ref, b_ref, o_ref):
    o_ref[...] = (
        jnp.dot(x_ref[...], w_ref[...], preferred_element_type=jnp.float32)
        + b_ref[...]
    )


def _matmul(x, w, b):
    m, k = x.shape
    n = w.shape[1]
    bm = 1000
    return pl.pallas_call(
        _matmul_kernel,
        grid=(m // bm,),
        in_specs=[
            pl.BlockSpec((bm, k), lambda i: (i, 0)),
            pl.BlockSpec((k, n), lambda i: (0, 0)),
            pl.BlockSpec((n,), lambda i: (0,)),
        ],
        out_specs=pl.BlockSpec((bm, n), lambda i: (i, 0)),
        out_shape=jax.ShapeDtypeStruct((m, n), jnp.float32),
    )(x, w, b)


def _sc_edge_body(xl, xr, srcs, dsts, wb, attv, out,
                  wb_v, att_v, idx_src, idx_dst, xl_buf, xr_buf, acc, asum,
                  sem_g):
    w = lax.axis_index("s") * NC + lax.axis_index("c")
    nd0 = w * NPW

    pltpu.sync_copy(wb.at[w], wb_v)
    pltpu.sync_copy(attv, att_v)

    wrow = wb_v[...]
    e0 = wrow[0]
    e1 = wrow[1]
    a0 = pl.multiple_of((e0 // 8) * 8, 8)
    nch = (e1 - a0 + (K - 1)) // K

    att_regs = [att_v[pl.ds(LANES * i, LANES)] for i in range(NBLK)]

    def issue(ci, buf):
        base = pl.multiple_of(a0 + ci * K, 8)
        pltpu.sync_copy(srcs.at[pl.ds(base, K)], idx_src.at[buf])
        pltpu.sync_copy(dsts.at[pl.ds(base, K)], idx_dst.at[buf])
        pltpu.make_async_copy(
            xl.at[idx_src.at[buf]], xl_buf.at[buf], sem_g.at[buf, 0]).start()
        pltpu.make_async_copy(
            xr.at[idx_dst.at[buf]], xr_buf.at[buf], sem_g.at[buf, 1]).start()

    zero = jnp.zeros((LANES,), jnp.float32)

    def zero_body(k, c):
        koff = pl.multiple_of(k * HID, 8)
        for i in range(NBLK):
            acc[pl.ds(koff + LANES * i, LANES)] = zero
        asum[pl.ds(pl.multiple_of(k * LANES, 8), LANES)] = zero
        return c
    lax.fori_loop(0, NPW, zero_body, 0)

    issue(0, 0)
    issue(1, 1)

    def chunk_body(ci, carry):
        buf = lax.rem(ci, 2)
        base = a0 + ci * K
        pltpu.make_async_copy(
            xl.at[idx_src.at[buf]], xl_buf.at[buf], sem_g.at[buf, 0]).wait()
        pltpu.make_async_copy(
            xr.at[idx_dst.at[buf]], xr_buf.at[buf], sem_g.at[buf, 1]).wait()

        def group_body(g, carry2):
            dvec = idx_dst[buf, pl.ds(LANES * g, LANES)]
            for jj in range(LANES):
                j = LANES * g + jj
                e = base + j
                valid = jnp.logical_and(e >= e0, e < e1)
                lv = jnp.where(valid, dvec[jj] - nd0, 0)
                aoff = pl.multiple_of(lv * HID, 8)

                xlr = [xl_buf[buf, j, pl.ds(LANES * i, LANES)]
                       for i in range(NBLK)]
                xrr = [xr_buf[buf, j, pl.ds(LANES * i, LANES)]
                       for i in range(NBLK)]
                t = []
                for i in range(NBLK):
                    m = xlr[i] + xrr[i]
                    ee = jnp.maximum(m, 0.2 * m)
                    t.append(ee * att_regs[i])
                while len(t) > 1:
                    t = [t[2 * i2] + t[2 * i2 + 1]
                         for i2 in range(len(t) // 2)]
                alpha = t[0] + jnp.flip(t[0])
                wv = jnp.exp(alpha) * jnp.where(valid, 1.0, 0.0)
                for i in range(NBLK):
                    plsc.addupdate(
                        acc.at[pl.ds(aoff + LANES * i, LANES)], wv * xlr[i])
                plsc.addupdate(
                    asum.at[pl.ds(pl.multiple_of(lv * LANES, 8), LANES)], wv)
            return carry2

        carry = lax.fori_loop(0, K // LANES, group_body, carry)

        @pl.when(ci + 2 < nch)
        def _issue_next():
            issue(ci + 2, buf)

        return carry

    lax.fori_loop(0, nch, chunk_body, 0)

    def norm_body(k, c):
        koff = pl.multiple_of(k * HID, 8)
        a = asum[pl.ds(pl.multiple_of(k * LANES, 8), LANES)]
        inv = 1.0 / (a + 1e-16)
        for i in range(NBLK):
            off = koff + LANES * i
            acc[pl.ds(off, LANES)] = acc[pl.ds(off, LANES)] * inv
        return c
    lax.fori_loop(0, NPW, norm_body, 0)
    pltpu.sync_copy(acc, out.at[pl.ds(nd0 * HID, NPW * HID)])


@functools.partial(
    pl.kernel,
    mesh=plsc.VectorSubcoreMesh(core_axis_name="c", subcore_axis_name="s"),
    out_type=jax.ShapeDtypeStruct((NP * HID,), jnp.float32),
    compiler_params=pltpu.CompilerParams(needs_layout_passes=False),
    scratch_types=[
        pltpu.VMEM((LANES,), jnp.int32),
        pltpu.VMEM((HID,), jnp.float32),
        pltpu.VMEM((2, K), jnp.int32),
        pltpu.VMEM((2, K), jnp.int32),
        pltpu.VMEM((2, K, HID), jnp.float32),
        pltpu.VMEM((2, K, HID), jnp.float32),
        pltpu.VMEM((NPW * HID,), jnp.float32),
        pltpu.VMEM((NPW * LANES,), jnp.float32),
        pltpu.SemaphoreType.DMA((2, 2)),
    ],
)
def _sc_edge(xl, xr, srcs, dsts, wb, attv, out,
             wb_v, att_v, idx_src, idx_dst, xl_buf, xr_buf, acc, asum,
             sem_g):
    _sc_edge_body(xl, xr, srcs, dsts, wb, attv, out,
                  wb_v, att_v, idx_src, idx_dst, xl_buf, xr_buf, acc, asum,
                  sem_g)


def kernel(x, edge_index, W_in, b_in, bn_in_g, bn_in_b, Wl, bl, Wr, br, att,
           gat_bias, ln_g, ln_b, scales, scale_weights, W1, b1, bn1_g, bn1_b,
           W2, b2, W3, b3):
    loop = jnp.arange(N, dtype=edge_index.dtype)
    src = jnp.concatenate([edge_index[0], loop]).astype(jnp.int32)
    dst = jnp.concatenate([edge_index[1], loop]).astype(jnp.int32)
    perm = jnp.argsort(dst)
    srcs = src[perm]
    dsts = dst[perm]
    srcs_p = jnp.pad(srcs, (0, E2P - E2))
    dsts_p = jnp.pad(dsts, (0, E2P - E2))
    wb = jnp.searchsorted(dsts, jnp.arange(33, dtype=jnp.int32) * NPW
                          ).astype(jnp.int32)
    wtab = jnp.zeros((NW, LANES), jnp.int32)
    wtab = wtab.at[:, 0].set(wb[:NW]).at[:, 1].set(wb[1:NW + 1])

    sidx = jnp.asarray(_SIDX)
    S = jax.nn.one_hot(sidx, HID, axis=0, dtype=jnp.float32)
    W_in_p = W_in @ S
    Wl_pp = jnp.einsum('sp,lst,tq->lpq', S, Wl, S)
    Wr_pp = jnp.einsum('sp,lst,tq->lpq', S, Wr, S)
    W1_pp = S.T @ W1

    h = _leaky(_batchnorm(_matmul(x, W_in_p, b_in[sidx]),
                          bn_in_g[sidx], bn_in_b[sidx]))
    w = jax.nn.softmax(scale_weights)
    hacc = jnp.zeros_like(h)
    for i in range(L):
        hn = _layernorm(h, ln_g[i][sidx], ln_b[i][sidx])
        xl = _matmul(hn, Wl_pp[i], bl[i][sidx])
        xr = _matmul(hn, Wr_pp[i], br[i][sidx])
        gat = _sc_edge(xl, xr, srcs_p, dsts_p, wtab, att[i].reshape(-1)[sidx])
        hg = _leaky(gat.reshape(NP, HID)[:N] + gat_bias[i][sidx])
        h = h + scales[i] * hg
        hacc = hacc + w[i] * h
    h = _leaky(_batchnorm(_matmul(hacc, W1_pp, b1), bn1_g, bn1_b))
    h = _leaky(h @ W2 + b2)
    return h @ W3 + b3


# speedup vs baseline: 3.3478x; 1.0223x over previous
import functools

import jax
import jax.numpy as jnp
import numpy as np
from jax import lax
from jax.experimental import pallas as pl
from jax.experimental.pallas import tpu as pltpu
from jax.experimental.pallas import tpu_sc as plsc

N = 10000
E = 320000
DIN = 128
HID = 256
HEADS = 8
FP = 32
L = 6

NC, NS, LANES = 2, 16, 16
NW = NC * NS
NPW = 320
NP = NW * NPW
K = 32
E2 = E + N
E2P = E2 + 240
NBLK = HID // LANES

_SIDX = np.zeros((HID,), np.int32)
for _p in range(HID):
    _i, _l = _p // LANES, _p % LANES
    _f = 2 * _i if _l < 8 else 2 * _i + 1
    _h = _l if _l < 8 else 15 - _l
    _SIDX[_p] = _h * FP + _f
_INV_SIDX = np.argsort(_SIDX).astype(np.int32)


def _leaky(x, s=0.1):
    return jnp.where(x > 0, x, s * x)


def _batchnorm(x, g, b):
    m = jnp.mean(x, axis=0)
    v = jnp.var(x, axis=0)
    return (x - m) / jnp.sqrt(v + 1e-5) * g + b


def _layernorm(x, g, b):
    m = jnp.mean(x, axis=-1, keepdims=True)
    v = jnp.var(x, axis=-1, keepdims=True)
    return (x - m) / jnp.sqrt(v + 1e-5) * g + b


def _matmul_kernel(x_ref, w_ref, b_ref, o_ref):
    o_ref[...] = (
        jnp.dot(x_ref[...], w_ref[...], preferred_element_type=jnp.float32)
        + b_ref[...]
    )


def _matmul(x, w, b):
    m, k = x.shape
    n = w.shape[1]
    bm = 1000
    return pl.pallas_call(
        _matmul_kernel,
        grid=(m // bm,),
        in_specs=[
            pl.BlockSpec((bm, k), lambda i: (i, 0)),
            pl.BlockSpec((k, n), lambda i: (0, 0)),
            pl.BlockSpec((n,), lambda i: (0,)),
        ],
        out_specs=pl.BlockSpec((bm, n), lambda i: (i, 0)),
        out_shape=jax.ShapeDtypeStruct((m, n), jnp.float32),
    )(x, w, b)


def _sc_edge_body(xl, xr, srcs, dsts, wb, attv, out,
                  wb_v, att_v, idx_src, idx_dst, xl_buf, xr_buf, acc, asum,
                  sem_g, sem_i):
    w = lax.axis_index("s") * NC + lax.axis_index("c")
    nd0 = w * NPW

    pltpu.sync_copy(wb.at[w], wb_v)
    pltpu.sync_copy(attv, att_v)

    wrow = wb_v[...]
    e0 = wrow[0]
    e1 = wrow[1]
    a0 = pl.multiple_of((e0 // 8) * 8, 8)
    nch = (e1 - a0 + (K - 1)) // K

    att_regs = [att_v[pl.ds(LANES * i, LANES)] for i in range(NBLK)]

    def issue_idx(ci):
        buf = lax.rem(ci, 2)
        base = pl.multiple_of(a0 + ci * K, 8)
        pltpu.make_async_copy(
            srcs.at[pl.ds(base, K)], idx_src.at[buf], sem_i.at[buf, 0]).start()
        pltpu.make_async_copy(
            dsts.at[pl.ds(base, K)], idx_dst.at[buf], sem_i.at[buf, 1]).start()

    def wait_idx(ci):
        buf = lax.rem(ci, 2)
        pltpu.make_async_copy(
            srcs.at[pl.ds(a0, K)], idx_src.at[buf], sem_i.at[buf, 0]).wait()
        pltpu.make_async_copy(
            dsts.at[pl.ds(a0, K)], idx_dst.at[buf], sem_i.at[buf, 1]).wait()

    def issue_gather(ci):
        buf = lax.rem(ci, 2)
        pltpu.make_async_copy(
            xl.at[idx_src.at[buf]], xl_buf.at[buf], sem_g.at[buf, 0]).start()
        pltpu.make_async_copy(
            xr.at[idx_dst.at[buf]], xr_buf.at[buf], sem_g.at[buf, 1]).start()

    zero = jnp.zeros((LANES,), jnp.float32)

    def zero_body(k, c):
        koff = pl.multiple_of(k * HID, 8)
        for i in range(NBLK):
            acc[pl.ds(koff + LANES * i, LANES)] = zero
        asum[pl.ds(pl.multiple_of(k * LANES, 8), LANES)] = zero
        return c
    lax.fori_loop(0, NPW, zero_body, 0)

    issue_idx(0)
    issue_idx(1)
    wait_idx(0)
    issue_gather(0)

    def chunk_body(ci, carry):
        buf = lax.rem(ci, 2)
        base = a0 + ci * K
        pltpu.make_async_copy(
            xl.at[idx_src.at[buf]], xl_buf.at[buf], sem_g.at[buf, 0]).wait()
        pltpu.make_async_copy(
            xr.at[idx_dst.at[buf]], xr_buf.at[buf], sem_g.at[buf, 1]).wait()

        @pl.when(ci + 1 < nch)
        def _issue_gather_next():
            wait_idx(ci + 1)
            issue_gather(ci + 1)

        def group_body(g, carry2):
            dvec = idx_dst[buf, pl.ds(LANES * g, LANES)]
            for jj in range(LANES):
                j = LANES * g + jj
                e = base + j
                valid = jnp.logical_and(e >= e0, e < e1)
                lv = jnp.where(valid, dvec[jj] - nd0, 0)
                aoff = pl.multiple_of(lv * HID, 8)

                xlr = [xl_buf[buf, j, pl.ds(LANES * i, LANES)]
                       for i in range(NBLK)]
                xrr = [xr_buf[buf, j, pl.ds(LANES * i, LANES)]
                       for i in range(NBLK)]
                t = []
                for i in range(NBLK):
                    m = xlr[i] + xrr[i]
                    ee = jnp.maximum(m, 0.2 * m)
                    t.append(ee * att_regs[i])
                while len(t) > 1:
                    t = [t[2 * i2] + t[2 * i2 + 1]
                         for i2 in range(len(t) // 2)]
                alpha = t[0] + jnp.flip(t[0])
                wv = jnp.exp(alpha) * jnp.where(valid, 1.0, 0.0)
                for i in range(NBLK):
                    plsc.addupdate(
                        acc.at[pl.ds(aoff + LANES * i, LANES)], wv * xlr[i])
                plsc.addupdate(
                    asum.at[pl.ds(pl.multiple_of(lv * LANES, 8), LANES)], wv)
            return carry2

        carry = lax.fori_loop(0, K // LANES, group_body, carry)

        @pl.when(ci + 2 < nch)
        def _issue_idx_next():
            issue_idx(ci + 2)

        return carry

    lax.fori_loop(0, nch, chunk_body, 0)

    def norm_body(k, c):
        koff = pl.multiple_of(k * HID, 8)
        a = asum[pl.ds(pl.multiple_of(k * LANES, 8), LANES)]
        inv = 1.0 / (a + 1e-16)
        for i in range(NBLK):
            off = koff + LANES * i
            acc[pl.ds(off, LANES)] = acc[pl.ds(off, LANES)] * inv
        return c
    lax.fori_loop(0, NPW, norm_body, 0)
    pltpu.sync_copy(acc, out.at[pl.ds(nd0 * HID, NPW * HID)])


@functools.partial(
    pl.kernel,
    mesh=plsc.VectorSubcoreMesh(core_axis_name="c", subcore_axis_name="s"),
    out_type=jax.ShapeDtypeStruct((NP * HID,), jnp.float32),
    compiler_params=pltpu.CompilerParams(needs_layout_passes=False),
    scratch_types=[
        pltpu.VMEM((LANES,), jnp.int32),
        pltpu.VMEM((HID,), jnp.float32),
        pltpu.VMEM((2, K), jnp.int32),
        pltpu.VMEM((2, K), jnp.int32),
        pltpu.VMEM((2, K, HID), jnp.float32),
        pltpu.VMEM((2, K, HID), jnp.float32),
        pltpu.VMEM((NPW * HID,), jnp.float32),
        pltpu.VMEM((NPW * LANES,), jnp.float32),
        pltpu.SemaphoreType.DMA((2, 2)),
        pltpu.SemaphoreType.DMA((2, 2)),
    ],
)
def _sc_edge(xl, xr, srcs, dsts, wb, attv, out,
             wb_v, att_v, idx_src, idx_dst, xl_buf, xr_buf, acc, asum,
             sem_g, sem_i):
    _sc_edge_body(xl, xr, srcs, dsts, wb, attv, out,
                  wb_v, att_v, idx_src, idx_dst, xl_buf, xr_buf, acc, asum,
                  sem_g, sem_i)


def kernel(x, edge_index, W_in, b_in, bn_in_g, bn_in_b, Wl, bl, Wr, br, att,
           gat_bias, ln_g, ln_b, scales, scale_weights, W1, b1, bn1_g, bn1_b,
           W2, b2, W3, b3):
    loop = jnp.arange(N, dtype=edge_index.dtype)
    src = jnp.concatenate([edge_index[0], loop]).astype(jnp.int32)
    dst = jnp.concatenate([edge_index[1], loop]).astype(jnp.int32)
    perm = jnp.argsort(dst)
    srcs = src[perm]
    dsts = dst[perm]
    srcs_p = jnp.pad(srcs, (0, E2P - E2))
    dsts_p = jnp.pad(dsts, (0, E2P - E2))
    wb = jnp.searchsorted(dsts, jnp.arange(33, dtype=jnp.int32) * NPW
                          ).astype(jnp.int32)
    wtab = jnp.zeros((NW, LANES), jnp.int32)
    wtab = wtab.at[:, 0].set(wb[:NW]).at[:, 1].set(wb[1:NW + 1])

    sidx = jnp.asarray(_SIDX)
    S = jax.nn.one_hot(sidx, HID, axis=0, dtype=jnp.float32)
    W_in_p = W_in @ S
    Wl_pp = jnp.einsum('sp,lst,tq->lpq', S, Wl, S)
    Wr_pp = jnp.einsum('sp,lst,tq->lpq', S, Wr, S)
    W1_pp = S.T @ W1

    h = _leaky(_batchnorm(_matmul(x, W_in_p, b_in[sidx]),
                          bn_in_g[sidx], bn_in_b[sidx]))
    w = jax.nn.softmax(scale_weights)
    hacc = jnp.zeros_like(h)
    for i in range(L):
        hn = _layernorm(h, ln_g[i][sidx], ln_b[i][sidx])
        xl = _matmul(hn, Wl_pp[i], bl[i][sidx])
        xr = _matmul(hn, Wr_pp[i], br[i][sidx])
        gat = _sc_edge(xl, xr, srcs_p, dsts_p, wtab, att[i].reshape(-1)[sidx])
        hg = _leaky(gat.reshape(NP, HID)[:N] + gat_bias[i][sidx])
        h = h + scales[i] * hg
        hacc = hacc + w[i] * h
    h = _leaky(_batchnorm(_matmul(hacc, W1_pp, b1), bn1_g, bn1_b))
    h = _leaky(h @ W2 + b2)
    return h @ W3 + b3
